# baseline (speedup 1.0000x reference)
import functools
import math

import jax
import jax.numpy as jnp
from jax import lax
from jax.experimental import pallas as pl
from jax.experimental.pallas import tpu as pltpu
from jax.experimental.pallas import tpu_sc as plsc

EMB = 64
SCALE = math.sqrt(EMB)
LANES = 16

_CHUNK = 128


@functools.cache
def _build(B: int):
    info = plsc.get_sparse_core_info()
    NC, NS = info.num_cores, info.num_subcores
    NW = NC * NS
    assert B % (NW * _CHUNK) == 0
    b_per_w = B // NW
    n_chunks = b_per_w // _CHUNK
    mesh = plsc.VectorSubcoreMesh(core_axis_name="c", subcore_axis_name="s")

    @functools.partial(
        pl.kernel,
        mesh=mesh,
        out_type=jax.ShapeDtypeStruct((B, EMB), jnp.float32),
        scratch_types=[
            pltpu.VMEM((_CHUNK,), jnp.int32),
            pltpu.VMEM((_CHUNK, EMB), jnp.float32),
            pltpu.SemaphoreType.DMA,
        ],
        compiler_params=pltpu.CompilerParams(use_tc_tiling_on_sc=False),
    )
    def k(tok_hbm, table_hbm, out_hbm, idx_v, rows_v, sem):
        wid = lax.axis_index("s") * NC + lax.axis_index("c")
        base0 = wid * b_per_w

        def body(g, carry):
            base = base0 + g * _CHUNK
            pltpu.sync_copy(tok_hbm.at[pl.ds(base, _CHUNK)], idx_v)
            pltpu.async_copy(table_hbm.at[idx_v], rows_v, sem).wait()

            def scale_row(r, c):
                for j in range(EMB // LANES):
                    sl = pl.ds(j * LANES, LANES)
                    rows_v[r, sl] = rows_v[r, sl] * SCALE
                return c

            lax.fori_loop(0, _CHUNK, scale_row, 0)
            pltpu.sync_copy(rows_v, out_hbm.at[pl.ds(base, _CHUNK)])
            return carry

        lax.fori_loop(0, n_chunks, body, 0)

    return k


def kernel(tokens, table):
    B0, B1 = tokens.shape
    B = B0 * B1
    flat = tokens.reshape(B).astype(jnp.int32)
    out = _build(B)(flat, table)
    return out.reshape(B0, B1, EMB)


# speedup vs baseline: 1.2103x; 1.2103x over previous
import functools
import math

import jax
import jax.numpy as jnp
from jax import lax
from jax.experimental import pallas as pl
from jax.experimental.pallas import tpu as pltpu
from jax.experimental.pallas import tpu_sc as plsc

EMB = 64
SCALE = math.sqrt(EMB)
LANES = 16

_IDXW = 128
_QUADS = 4
_CHUNK = _IDXW * _QUADS


@functools.cache
def _build(B: int):
    info = plsc.get_sparse_core_info()
    NC, NS = info.num_cores, info.num_subcores
    NW = NC * NS
    assert B % (NW * _CHUNK) == 0
    b_per_w = B // NW
    n_chunks = b_per_w // _CHUNK
    assert n_chunks % 2 == 0 and n_chunks >= 4
    mesh = plsc.VectorSubcoreMesh(core_axis_name="c", subcore_axis_name="s")

    @functools.partial(
        pl.kernel,
        mesh=mesh,
        out_type=jax.ShapeDtypeStruct((B, EMB), jnp.float32),
        scratch_types=[
            pltpu.VMEM((2, _QUADS, _IDXW), jnp.int32),
            pltpu.VMEM((2, _CHUNK, EMB), jnp.float32),
            pltpu.SemaphoreType.DMA,
            pltpu.SemaphoreType.DMA,
            pltpu.SemaphoreType.DMA,
            pltpu.SemaphoreType.DMA,
        ],
        compiler_params=pltpu.CompilerParams(use_tc_tiling_on_sc=False),
    )
    def k(tok_hbm, table_hbm, out_hbm, idx_v, rows_v, g0, g1, o0, o1):
        wid = lax.axis_index("s") * NC + lax.axis_index("c")
        base0 = wid * b_per_w
        gsem = (g0, g1)
        osem = (o0, o1)

        def issue(g, b, first):
            if not first:
                pltpu.make_async_copy(
                    rows_v.at[b], out_hbm.at[pl.ds(base0, _CHUNK)], osem[b]
                ).wait()
            base = base0 + g * _CHUNK
            for q in range(_QUADS):
                pltpu.sync_copy(
                    tok_hbm.at[pl.ds(base + q * _IDXW, _IDXW)],
                    idx_v.at[b, q],
                )
            for q in range(_QUADS):
                pltpu.async_copy(
                    table_hbm.at[idx_v.at[b, q]],
                    rows_v.at[b, pl.ds(q * _IDXW, _IDXW)],
                    gsem[b],
                )

        def finish(g, b):
            for q in range(_QUADS):
                pltpu.make_async_copy(
                    table_hbm.at[idx_v.at[b, q]],
                    rows_v.at[b, pl.ds(q * _IDXW, _IDXW)],
                    gsem[b],
                ).wait()

            @functools.partial(plsc.parallel_loop, 0, _CHUNK, unroll=4)
            def _(r):
                for j in range(EMB // LANES):
                    sl = pl.ds(j * LANES, LANES)
                    rows_v[b, r, sl] = rows_v[b, r, sl] * SCALE

            base = base0 + g * _CHUNK
            pltpu.async_copy(rows_v.at[b], out_hbm.at[pl.ds(base, _CHUNK)], osem[b])

        issue(0, 0, True)
        issue(1, 1, True)
        finish(0, 0)
        issue(2, 0, False)
        finish(1, 1)

        def body(gg, carry):
            issue(2 * gg + 1, 1, False)
            finish(2 * gg, 0)
            issue(2 * gg + 2, 0, False)
            finish(2 * gg + 1, 1)
            return carry

        lax.fori_loop(1, n_chunks // 2 - 1, body, 0)

        issue(n_chunks - 1, 1, False)
        finish(n_chunks - 2, 0)
        finish(n_chunks - 1, 1)
        for b in range(2):
            pltpu.make_async_copy(
                rows_v.at[b], out_hbm.at[pl.ds(base0, _CHUNK)], osem[b]
            ).wait()

    return k


def kernel(tokens, table):
    B0, B1 = tokens.shape
    B = B0 * B1
    flat = tokens.reshape(B).astype(jnp.int32)
    out = _build(B)(flat, table)
    return out.reshape(B0, B1, EMB)
